# baseline (speedup 1.0000x reference)
import functools

import jax
import jax.numpy as jnp
from jax import lax
from jax.experimental import pallas as pl
from jax.experimental.pallas import tpu as pltpu
from jax.experimental.pallas import tpu_sc as plsc

B, V, D = 16384, 100000, 128
CHUNK = 128
NCHUNKS = 2


def _sc_gather(emb, idx2d, nidx, *, nc, ns):
    nw = nc * ns
    rows_per_w = nidx // nw
    chunks_per_w = rows_per_w // CHUNK
    idx_rows_per_w = rows_per_w // CHUNK

    mesh = plsc.VectorSubcoreMesh(
        core_axis_name="c", subcore_axis_name="s",
        num_cores=nc, num_subcores=ns)

    nbuf = 4
    depth = 2

    @functools.partial(
        pl.kernel,
        out_type=jax.ShapeDtypeStruct((nidx, D), jnp.float32),
        mesh=mesh,
        scratch_types=[
            pltpu.VMEM((idx_rows_per_w, CHUNK), jnp.int32),
        ] + [pltpu.VMEM((CHUNK, D), jnp.float32) for _ in range(nbuf)]
          + [pltpu.SemaphoreType.DMA for _ in range(2 * nbuf)],
    )
    def gather_kernel(emb_hbm, idx_hbm, out_hbm, idx_v, *rest):
        bufs = rest[:nbuf]
        gsems = rest[nbuf:2 * nbuf]
        wsems = rest[2 * nbuf:]
        wid = lax.axis_index("s") * nc + lax.axis_index("c")
        base_row = wid * rows_per_w
        pltpu.sync_copy(idx_hbm.at[pl.ds(wid * idx_rows_per_w, idx_rows_per_w)],
                        idx_v)

        def out_ref(j):
            return out_hbm.at[pl.ds(base_row + j * CHUNK, CHUNK)]

        for j in range(min(depth, chunks_per_w)):
            pltpu.async_copy(emb_hbm.at[idx_v.at[j]], bufs[j % nbuf],
                             gsems[j % nbuf])
        for j in range(chunks_per_w):
            k = j % nbuf
            pltpu.make_async_copy(emb_hbm.at[idx_v.at[j]], bufs[k],
                                  gsems[k]).wait()
            pltpu.async_copy(bufs[k], out_ref(j), wsems[k])
            jj = j + depth
            if jj < chunks_per_w:
                kk = jj % nbuf
                if jj >= nbuf:
                    pltpu.make_async_copy(bufs[kk], out_ref(jj - nbuf),
                                          wsems[kk]).wait()
                pltpu.async_copy(emb_hbm.at[idx_v.at[jj]], bufs[kk],
                                 gsems[kk])
        for j in range(max(0, chunks_per_w - nbuf), chunks_per_w):
            k = j % nbuf
            pltpu.make_async_copy(bufs[k], out_ref(j), wsems[k]).wait()

    return gather_kernel(emb, idx2d)


def _tc_mlp_kernel(g0_ref, g1_ref, w1a_ref, w1b_ref, b1_ref, w2_ref, b2_ref,
                   w3_ref, b3_ref, o_ref):
    h = (jnp.dot(g0_ref[...], w1a_ref[...], preferred_element_type=jnp.float32)
         + jnp.dot(g1_ref[...], w1b_ref[...],
                   preferred_element_type=jnp.float32))
    h = jnp.maximum(h + b1_ref[...], 0.0)
    h = jnp.dot(h, w2_ref[...], preferred_element_type=jnp.float32)
    h = jnp.maximum(h + b2_ref[...], 0.0)
    o_ref[...] = (jnp.sum(h * w3_ref[...], axis=1, keepdims=True)
                  + b3_ref[...])


def _tc_mlp(g, w1at, w1bt, b1r, w2t, b2r, w3r, b3r, *, bsz, blk):
    grid = (bsz // blk,)
    half = bsz // blk
    return pl.pallas_call(
        _tc_mlp_kernel,
        grid=grid,
        in_specs=[
            pl.BlockSpec((blk, D), lambda i: (i, 0)),
            pl.BlockSpec((blk, D), lambda i, h=half: (i + h, 0)),
            pl.BlockSpec((D, D), lambda i: (0, 0)),
            pl.BlockSpec((D, D), lambda i: (0, 0)),
            pl.BlockSpec((1, D), lambda i: (0, 0)),
            pl.BlockSpec((D, D), lambda i: (0, 0)),
            pl.BlockSpec((1, D), lambda i: (0, 0)),
            pl.BlockSpec((1, D), lambda i: (0, 0)),
            pl.BlockSpec((1, 1), lambda i: (0, 0)),
        ],
        out_specs=pl.BlockSpec((blk, 1), lambda i: (i, 0)),
        out_shape=jax.ShapeDtypeStruct((bsz, 1), jnp.float32),
    )(g, g, w1at, w1bt, b1r, w2t, b2r, w3r, b3r)


def kernel(x, emb, W1, b1, W2, b2, W3, b3):
    info = plsc.get_sparse_core_info()
    nc, ns = info.num_cores, info.num_subcores
    w1t = W1.T
    w1at, w1bt = w1t[:D], w1t[D:]
    b1r, b2r = b1.reshape(1, D), b2.reshape(1, D)
    w2t = W2.T
    w3r, b3r = W3.reshape(1, D), b3.reshape(1, 1)
    xi = x.astype(jnp.int32)

    bc = B // NCHUNKS
    outs = []
    for c in range(NCHUNKS):
        xc = lax.slice_in_dim(xi, c * bc, (c + 1) * bc, axis=0)
        idx2d = xc.T.reshape(2 * bc // CHUNK, CHUNK)
        g = _sc_gather(emb, idx2d, 2 * bc, nc=nc, ns=ns)
        outs.append(_tc_mlp(g, w1at, w1bt, b1r, w2t, b2r, w3r, b3r,
                            bsz=bc, blk=8192))
    return jnp.concatenate(outs, axis=0)


# speedup vs baseline: 1.0447x; 1.0447x over previous
import functools

import jax
import jax.numpy as jnp
from jax import lax
from jax.experimental import pallas as pl
from jax.experimental.pallas import tpu as pltpu
from jax.experimental.pallas import tpu_sc as plsc

B, V, D = 16384, 100000, 128
CHUNK = 128
NCHUNKS = 1


def _sc_gather(emb, idx2d, nidx, *, nc, ns):
    nw = nc * ns
    rows_per_w = nidx // nw
    chunks_per_w = rows_per_w // CHUNK
    idx_rows_per_w = rows_per_w // CHUNK

    mesh = plsc.VectorSubcoreMesh(
        core_axis_name="c", subcore_axis_name="s",
        num_cores=nc, num_subcores=ns)

    nbuf = 4
    depth = 2

    @functools.partial(
        pl.kernel,
        out_type=jax.ShapeDtypeStruct((nidx, D), jnp.float32),
        mesh=mesh,
        scratch_types=[
            pltpu.VMEM((idx_rows_per_w, CHUNK), jnp.int32),
        ] + [pltpu.VMEM((CHUNK, D), jnp.float32) for _ in range(nbuf)]
          + [pltpu.SemaphoreType.DMA for _ in range(2 * nbuf)],
    )
    def gather_kernel(emb_hbm, idx_hbm, out_hbm, idx_v, *rest):
        bufs = rest[:nbuf]
        gsems = rest[nbuf:2 * nbuf]
        wsems = rest[2 * nbuf:]
        wid = lax.axis_index("s") * nc + lax.axis_index("c")
        base_row = wid * rows_per_w
        pltpu.sync_copy(idx_hbm.at[pl.ds(wid * idx_rows_per_w, idx_rows_per_w)],
                        idx_v)

        def out_ref(j):
            return out_hbm.at[pl.ds(base_row + j * CHUNK, CHUNK)]

        for j in range(min(depth, chunks_per_w)):
            pltpu.async_copy(emb_hbm.at[idx_v.at[j]], bufs[j % nbuf],
                             gsems[j % nbuf])
        for j in range(chunks_per_w):
            k = j % nbuf
            pltpu.make_async_copy(emb_hbm.at[idx_v.at[j]], bufs[k],
                                  gsems[k]).wait()
            pltpu.async_copy(bufs[k], out_ref(j), wsems[k])
            jj = j + depth
            if jj < chunks_per_w:
                kk = jj % nbuf
                if jj >= nbuf:
                    pltpu.make_async_copy(bufs[kk], out_ref(jj - nbuf),
                                          wsems[kk]).wait()
                pltpu.async_copy(emb_hbm.at[idx_v.at[jj]], bufs[kk],
                                 gsems[kk])
        for j in range(max(0, chunks_per_w - nbuf), chunks_per_w):
            k = j % nbuf
            pltpu.make_async_copy(bufs[k], out_ref(j), wsems[k]).wait()

    return gather_kernel(emb, idx2d)


def _tc_mlp_kernel(g0_ref, g1_ref, w1a_ref, w1b_ref, b1_ref, w2_ref, b2_ref,
                   w3_ref, b3_ref, o_ref):
    h = (jnp.dot(g0_ref[...], w1a_ref[...], preferred_element_type=jnp.float32)
         + jnp.dot(g1_ref[...], w1b_ref[...],
                   preferred_element_type=jnp.float32))
    h = jnp.maximum(h + b1_ref[...], 0.0)
    h = jnp.dot(h, w2_ref[...], preferred_element_type=jnp.float32)
    h = jnp.maximum(h + b2_ref[...], 0.0)
    o_ref[...] = (jnp.sum(h * w3_ref[...], axis=1, keepdims=True)
                  + b3_ref[...])


def _tc_mlp(g, w1at, w1bt, b1r, w2t, b2r, w3r, b3r, *, bsz, blk):
    grid = (bsz // blk,)
    half = bsz // blk
    return pl.pallas_call(
        _tc_mlp_kernel,
        grid=grid,
        in_specs=[
            pl.BlockSpec((blk, D), lambda i: (i, 0)),
            pl.BlockSpec((blk, D), lambda i, h=half: (i + h, 0)),
            pl.BlockSpec((D, D), lambda i: (0, 0)),
            pl.BlockSpec((D, D), lambda i: (0, 0)),
            pl.BlockSpec((1, D), lambda i: (0, 0)),
            pl.BlockSpec((D, D), lambda i: (0, 0)),
            pl.BlockSpec((1, D), lambda i: (0, 0)),
            pl.BlockSpec((1, D), lambda i: (0, 0)),
            pl.BlockSpec((1, 1), lambda i: (0, 0)),
        ],
        out_specs=pl.BlockSpec((blk, 1), lambda i: (i, 0)),
        out_shape=jax.ShapeDtypeStruct((bsz, 1), jnp.float32),
    )(g, g, w1at, w1bt, b1r, w2t, b2r, w3r, b3r)


def kernel(x, emb, W1, b1, W2, b2, W3, b3):
    info = plsc.get_sparse_core_info()
    nc, ns = info.num_cores, info.num_subcores
    w1t = W1.T
    w1at, w1bt = w1t[:D], w1t[D:]
    b1r, b2r = b1.reshape(1, D), b2.reshape(1, D)
    w2t = W2.T
    w3r, b3r = W3.reshape(1, D), b3.reshape(1, 1)
    xi = x.astype(jnp.int32)

    bc = B // NCHUNKS
    outs = []
    for c in range(NCHUNKS):
        xc = lax.slice_in_dim(xi, c * bc, (c + 1) * bc, axis=0)
        idx2d = xc.T.reshape(2 * bc // CHUNK, CHUNK)
        g = _sc_gather(emb, idx2d, 2 * bc, nc=nc, ns=ns)
        outs.append(_tc_mlp(g, w1at, w1bt, b1r, w2t, b2r, w3r, b3r,
                            bsz=bc, blk=4096))
    return jnp.concatenate(outs, axis=0)


# speedup vs baseline: 1.0484x; 1.0035x over previous
import functools

import jax
import jax.numpy as jnp
from jax import lax
from jax.experimental import pallas as pl
from jax.experimental.pallas import tpu as pltpu
from jax.experimental.pallas import tpu_sc as plsc

B, V, D = 16384, 100000, 128
CHUNK = 128
NCHUNKS = 1


def _sc_gather(emb, idx2d, nidx, *, nc, ns):
    nw = nc * ns
    rows_per_w = nidx // nw
    chunks_per_w = rows_per_w // CHUNK
    idx_rows_per_w = rows_per_w // CHUNK

    mesh = plsc.VectorSubcoreMesh(
        core_axis_name="c", subcore_axis_name="s",
        num_cores=nc, num_subcores=ns)

    nbuf = 6
    depth = 3

    @functools.partial(
        pl.kernel,
        out_type=jax.ShapeDtypeStruct((nidx, D), jnp.float32),
        mesh=mesh,
        scratch_types=[
            pltpu.VMEM((idx_rows_per_w, CHUNK), jnp.int32),
        ] + [pltpu.VMEM((CHUNK, D), jnp.float32) for _ in range(nbuf)]
          + [pltpu.SemaphoreType.DMA for _ in range(2 * nbuf)],
    )
    def gather_kernel(emb_hbm, idx_hbm, out_hbm, idx_v, *rest):
        bufs = rest[:nbuf]
        gsems = rest[nbuf:2 * nbuf]
        wsems = rest[2 * nbuf:]
        wid = lax.axis_index("s") * nc + lax.axis_index("c")
        base_row = wid * rows_per_w
        pltpu.sync_copy(idx_hbm.at[pl.ds(wid * idx_rows_per_w, idx_rows_per_w)],
                        idx_v)

        def out_ref(j):
            return out_hbm.at[pl.ds(base_row + j * CHUNK, CHUNK)]

        for j in range(min(depth, chunks_per_w)):
            pltpu.async_copy(emb_hbm.at[idx_v.at[j]], bufs[j % nbuf],
                             gsems[j % nbuf])
        for j in range(chunks_per_w):
            k = j % nbuf
            pltpu.make_async_copy(emb_hbm.at[idx_v.at[j]], bufs[k],
                                  gsems[k]).wait()
            pltpu.async_copy(bufs[k], out_ref(j), wsems[k])
            jj = j + depth
            if jj < chunks_per_w:
                kk = jj % nbuf
                if jj >= nbuf:
                    pltpu.make_async_copy(bufs[kk], out_ref(jj - nbuf),
                                          wsems[kk]).wait()
                pltpu.async_copy(emb_hbm.at[idx_v.at[jj]], bufs[kk],
                                 gsems[kk])
        for j in range(max(0, chunks_per_w - nbuf), chunks_per_w):
            k = j % nbuf
            pltpu.make_async_copy(bufs[k], out_ref(j), wsems[k]).wait()

    return gather_kernel(emb, idx2d)


def _tc_mlp_kernel(g0_ref, g1_ref, w1a_ref, w1b_ref, b1_ref, w2_ref, b2_ref,
                   w3_ref, b3_ref, o_ref):
    h = (jnp.dot(g0_ref[...], w1a_ref[...], preferred_element_type=jnp.float32)
         + jnp.dot(g1_ref[...], w1b_ref[...],
                   preferred_element_type=jnp.float32))
    h = jnp.maximum(h + b1_ref[...], 0.0)
    h = jnp.dot(h, w2_ref[...], preferred_element_type=jnp.float32)
    h = jnp.maximum(h + b2_ref[...], 0.0)
    o_ref[...] = (jnp.sum(h * w3_ref[...], axis=1, keepdims=True)
                  + b3_ref[...])


def _tc_mlp(g, w1at, w1bt, b1r, w2t, b2r, w3r, b3r, *, bsz, blk):
    grid = (bsz // blk,)
    half = bsz // blk
    return pl.pallas_call(
        _tc_mlp_kernel,
        grid=grid,
        in_specs=[
            pl.BlockSpec((blk, D), lambda i: (i, 0)),
            pl.BlockSpec((blk, D), lambda i, h=half: (i + h, 0)),
            pl.BlockSpec((D, D), lambda i: (0, 0)),
            pl.BlockSpec((D, D), lambda i: (0, 0)),
            pl.BlockSpec((1, D), lambda i: (0, 0)),
            pl.BlockSpec((D, D), lambda i: (0, 0)),
            pl.BlockSpec((1, D), lambda i: (0, 0)),
            pl.BlockSpec((1, D), lambda i: (0, 0)),
            pl.BlockSpec((1, 1), lambda i: (0, 0)),
        ],
        out_specs=pl.BlockSpec((blk, 1), lambda i: (i, 0)),
        out_shape=jax.ShapeDtypeStruct((bsz, 1), jnp.float32),
    )(g, g, w1at, w1bt, b1r, w2t, b2r, w3r, b3r)


def kernel(x, emb, W1, b1, W2, b2, W3, b3):
    info = plsc.get_sparse_core_info()
    nc, ns = info.num_cores, info.num_subcores
    w1t = W1.T
    w1at, w1bt = w1t[:D], w1t[D:]
    b1r, b2r = b1.reshape(1, D), b2.reshape(1, D)
    w2t = W2.T
    w3r, b3r = W3.reshape(1, D), b3.reshape(1, 1)
    xi = x.astype(jnp.int32)

    bc = B // NCHUNKS
    outs = []
    for c in range(NCHUNKS):
        xc = lax.slice_in_dim(xi, c * bc, (c + 1) * bc, axis=0)
        idx2d = xc.T.reshape(2 * bc // CHUNK, CHUNK)
        g = _sc_gather(emb, idx2d, 2 * bc, nc=nc, ns=ns)
        outs.append(_tc_mlp(g, w1at, w1bt, b1r, w2t, b2r, w3r, b3r,
                            bsz=bc, blk=4096))
    return jnp.concatenate(outs, axis=0)


# speedup vs baseline: 1.1984x; 1.1431x over previous
import functools

import jax
import jax.numpy as jnp
from jax import lax
from jax.experimental import pallas as pl
from jax.experimental.pallas import tpu as pltpu
from jax.experimental.pallas import tpu_sc as plsc

B, V, D = 16384, 100000, 128
CHUNK = 128
NCHUNKS = 1


def _sc_gather(emb, idx2d, nidx, *, nc, ns):
    nw = nc * ns
    rows_per_w = nidx // nw
    chunks_per_w = rows_per_w // CHUNK
    idx_rows_per_w = rows_per_w // CHUNK

    mesh = plsc.VectorSubcoreMesh(
        core_axis_name="c", subcore_axis_name="s",
        num_cores=nc, num_subcores=ns)

    nbuf = 6
    depth = 3

    @functools.partial(
        pl.kernel,
        out_type=jax.ShapeDtypeStruct((nidx, D), jnp.float32),
        mesh=mesh,
        scratch_types=[
            pltpu.VMEM((idx_rows_per_w, CHUNK), jnp.int32),
        ] + [pltpu.VMEM((CHUNK, D), jnp.float32) for _ in range(nbuf)]
          + [pltpu.SemaphoreType.DMA for _ in range(2 * nbuf)],
    )
    def gather_kernel(emb_hbm, idx_hbm, out_hbm, idx_v, *rest):
        bufs = rest[:nbuf]
        gsems = rest[nbuf:2 * nbuf]
        wsems = rest[2 * nbuf:]
        wid = lax.axis_index("s") * nc + lax.axis_index("c")
        base_row = wid * rows_per_w
        pltpu.sync_copy(idx_hbm.at[pl.ds(wid * idx_rows_per_w, idx_rows_per_w)],
                        idx_v)

        def out_ref(j):
            return out_hbm.at[pl.ds(base_row + j * CHUNK, CHUNK)]

        for j in range(min(depth, chunks_per_w)):
            pltpu.async_copy(emb_hbm.at[idx_v.at[j]], bufs[j % nbuf],
                             gsems[j % nbuf])
        for j in range(chunks_per_w):
            k = j % nbuf
            pltpu.make_async_copy(emb_hbm.at[idx_v.at[j]], bufs[k],
                                  gsems[k]).wait()
            pltpu.async_copy(bufs[k], out_ref(j), wsems[k])
            jj = j + depth
            if jj < chunks_per_w:
                kk = jj % nbuf
                if jj >= nbuf:
                    pltpu.make_async_copy(bufs[kk], out_ref(jj - nbuf),
                                          wsems[kk]).wait()
                pltpu.async_copy(emb_hbm.at[idx_v.at[jj]], bufs[kk],
                                 gsems[kk])
        for j in range(max(0, chunks_per_w - nbuf), chunks_per_w):
            k = j % nbuf
            pltpu.make_async_copy(bufs[k], out_ref(j), wsems[k]).wait()

    return gather_kernel(emb, idx2d)


def _tc_mlp_kernel(g0_ref, g1_ref, w1a_ref, w1b_ref, b1_ref, w2_ref, b2_ref,
                   w3_ref, b3_ref, o_ref):
    h = (jnp.dot(g0_ref[...], w1a_ref[...], preferred_element_type=jnp.float32)
         + jnp.dot(g1_ref[...], w1b_ref[...],
                   preferred_element_type=jnp.float32))
    h = jnp.maximum(h + b1_ref[...], 0.0)
    h = jnp.dot(h, w2_ref[...], preferred_element_type=jnp.float32)
    h = jnp.maximum(h + b2_ref[...], 0.0)
    o_ref[...] = lax.dot_general(
        w3_ref[...], h, (((1,), (1,)), ((), ())),
        preferred_element_type=jnp.float32) + b3_ref[...]


def _tc_mlp(g, w1at, w1bt, b1r, w2t, b2r, w3r, b3r, *, bsz, blk):
    grid = (bsz // blk,)
    half = bsz // blk
    return pl.pallas_call(
        _tc_mlp_kernel,
        grid=grid,
        in_specs=[
            pl.BlockSpec((blk, D), lambda i: (i, 0)),
            pl.BlockSpec((blk, D), lambda i, h=half: (i + h, 0)),
            pl.BlockSpec((D, D), lambda i: (0, 0)),
            pl.BlockSpec((D, D), lambda i: (0, 0)),
            pl.BlockSpec((1, D), lambda i: (0, 0)),
            pl.BlockSpec((D, D), lambda i: (0, 0)),
            pl.BlockSpec((1, D), lambda i: (0, 0)),
            pl.BlockSpec((1, D), lambda i: (0, 0)),
            pl.BlockSpec((1, 1), lambda i: (0, 0)),
        ],
        out_specs=pl.BlockSpec((1, blk), lambda i: (0, i)),
        out_shape=jax.ShapeDtypeStruct((1, bsz), jnp.float32),
    )(g, g, w1at, w1bt, b1r, w2t, b2r, w3r, b3r).reshape(bsz, 1)


def kernel(x, emb, W1, b1, W2, b2, W3, b3):
    info = plsc.get_sparse_core_info()
    nc, ns = info.num_cores, info.num_subcores
    w1t = W1.T
    w1at, w1bt = w1t[:D], w1t[D:]
    b1r, b2r = b1.reshape(1, D), b2.reshape(1, D)
    w2t = W2.T
    w3r, b3r = W3.reshape(1, D), b3.reshape(1, 1)
    xi = x.astype(jnp.int32)

    bc = B // NCHUNKS
    outs = []
    for c in range(NCHUNKS):
        xc = lax.slice_in_dim(xi, c * bc, (c + 1) * bc, axis=0)
        idx2d = xc.T.reshape(2 * bc // CHUNK, CHUNK)
        g = _sc_gather(emb, idx2d, 2 * bc, nc=nc, ns=ns)
        outs.append(_tc_mlp(g, w1at, w1bt, b1r, w2t, b2r, w3r, b3r,
                            bsz=bc, blk=4096))
    return jnp.concatenate(outs, axis=0)
